# Initial kernel scaffold; baseline (speedup 1.0000x reference)
import jax
import jax.numpy as jnp
from jax.experimental import pallas as pl


def kernel(x, edge_index, W, b):
    raise NotImplementedError("write your pallas kernel here")


# speedup vs baseline: 60.5126x; 60.5126x over previous
import functools

import jax
import jax.numpy as jnp
from jax import lax
from jax.experimental import pallas as pl
from jax.experimental.pallas import tpu as pltpu
from jax.experimental.pallas import tpu_sc as plsc

NN = 10000
EE = 320000
DD = 128
CC = 7
F = 8

NC = 2
NS = 16
NW = NC * NS
NP = 10240
CH = 128
EPW = 10240
NCH = EPW // CH
EPAD = NW * EPW
RPT = NP // NS

_mesh = plsc.VectorSubcoreMesh(
    core_axis_name="c", subcore_axis_name="s", num_cores=NC, num_subcores=NS)


def _deg_sc_body(col_hbm, ones_hbm, zeros_hbm, out_hbm, cidx_v, ones_v, acc_sh,
                 sem):
    cid = lax.axis_index("c")
    sid = lax.axis_index("s")
    wid = cid * NS + sid
    r0 = sid * RPT
    pltpu.sync_copy(zeros_hbm.at[pl.ds(r0, RPT)], acc_sh.at[pl.ds(r0, RPT)])
    pltpu.sync_copy(ones_hbm, ones_v)
    pltpu.sync_copy(col_hbm.at[wid], cidx_v)
    plsc.subcore_barrier()

    def body(g, carry):
        for b in range(8):
            pltpu.async_copy(ones_v, acc_sh.at[cidx_v.at[8 * g + b]], sem,
                             add=True)
        for b in range(8):
            pltpu.make_async_copy(ones_v, acc_sh.at[cidx_v.at[8 * g + b]],
                                  sem).wait()
        return carry

    lax.fori_loop(0, NCH // 8, body, 0)
    plsc.subcore_barrier()
    pltpu.sync_copy(acc_sh.at[pl.ds(r0, RPT)], out_hbm.at[cid, pl.ds(r0, RPT)])


def _make_deg_sc(interpret=False):
    return pl.kernel(
        _deg_sc_body,
        out_type=jax.ShapeDtypeStruct((NC, NP, F), jnp.float32),
        mesh=_mesh,
        scratch_types=[
            pltpu.VMEM((NCH, CH), jnp.int32),
            pltpu.VMEM((CH, F), jnp.float32),
            pltpu.VMEM_SHARED((NP, F), jnp.float32),
            pltpu.SemaphoreType.DMA,
        ],
        interpret=interpret,
        compiler_params=pltpu.CompilerParams(use_tc_tiling_on_sc=False),
    )


_deg_sc = _make_deg_sc()


def _hop_sc_body(u_hbm, row_hbm, col_hbm, zeros_hbm, out_hbm,
                 ridx_v, cidx_v, msg_v, acc_sh, u_sh, gsems, ssems):
    cid = lax.axis_index("c")
    sid = lax.axis_index("s")
    wid = cid * NS + sid
    r0 = sid * RPT
    pltpu.sync_copy(zeros_hbm.at[pl.ds(r0, RPT)], acc_sh.at[pl.ds(r0, RPT)])
    pltpu.sync_copy(u_hbm.at[pl.ds(r0, RPT)], u_sh.at[pl.ds(r0, RPT)])
    pltpu.sync_copy(row_hbm.at[wid], ridx_v)
    pltpu.sync_copy(col_hbm.at[wid], cidx_v)
    plsc.subcore_barrier()

    for b in range(4):
        pltpu.async_copy(u_sh.at[ridx_v.at[b]], msg_v.at[b], gsems.at[b])

    def body(k, carry):
        j0 = 4 * k
        for b in range(4):
            pltpu.make_async_copy(u_sh.at[ridx_v.at[j0 + b]], msg_v.at[b],
                                  gsems.at[b]).wait()
            pltpu.async_copy(msg_v.at[b], acc_sh.at[cidx_v.at[j0 + b]],
                             ssems.at[b], add=True)
        for b in range(4):
            pltpu.make_async_copy(msg_v.at[b], acc_sh.at[cidx_v.at[j0 + b]],
                                  ssems.at[b]).wait()

            @pl.when(j0 + b + 4 < NCH)
            def _():
                pltpu.async_copy(u_sh.at[ridx_v.at[j0 + b + 4]], msg_v.at[b],
                                 gsems.at[b])

        return carry

    lax.fori_loop(0, NCH // 4, body, 0)
    plsc.subcore_barrier()
    pltpu.sync_copy(acc_sh.at[pl.ds(r0, RPT)], out_hbm.at[cid, pl.ds(r0, RPT)])


def _make_hop_sc(interpret=False):
    return pl.kernel(
        _hop_sc_body,
        out_type=jax.ShapeDtypeStruct((NC, NP, F), jnp.float32),
        mesh=_mesh,
        scratch_types=[
            pltpu.VMEM((NCH, CH), jnp.int32),
            pltpu.VMEM((NCH, CH), jnp.int32),
            pltpu.VMEM((4, CH, F), jnp.float32),
            pltpu.VMEM_SHARED((NP, F), jnp.float32),
            pltpu.VMEM_SHARED((NP, F), jnp.float32),
            pltpu.SemaphoreType.DMA((4,)),
            pltpu.SemaphoreType.DMA((4,)),
        ],
        interpret=interpret,
        compiler_params=pltpu.CompilerParams(use_tc_tiling_on_sc=False),
    )


_hop_sc = _make_hop_sc()


def _tc_pre_body(x_ref, w_ref, degp_ref, u0_ref, d8_ref):
    z0 = jnp.dot(x_ref[...], w_ref[...], preferred_element_type=jnp.float32)
    deg = degp_ref[0] + degp_ref[1] + 1.0
    dinv = lax.rsqrt(deg)
    d8_ref[...] = dinv
    u0_ref[...] = jnp.zeros((NP, F), jnp.float32)
    u0_ref[0:NN, :] = dinv[0:NN, :] * z0


def _tc_mid_body(acc_ref, u0_ref, d8_ref, u1_ref):
    d = d8_ref[...]
    u1_ref[...] = d * d * (acc_ref[0] + acc_ref[1] + u0_ref[...])


def _tc_post_body(acc_ref, u1_ref, d8_ref, b_ref, out_ref):
    z2 = d8_ref[...] * (acc_ref[0] + acc_ref[1] + u1_ref[...])
    t = z2 + b_ref[...]
    col = lax.broadcasted_iota(jnp.int32, t.shape, 1)
    pad = col >= CC
    tm = jnp.where(pad, -jnp.inf, t)
    m = jnp.max(tm, axis=1, keepdims=True)
    e = jnp.where(pad, 0.0, jnp.exp(t - m))
    s = jnp.sum(e, axis=1, keepdims=True)
    out_ref[...] = t - m - jnp.log(s)


def kernel(x, edge_index, W, b):
    row = edge_index[0]
    col = edge_index[1]
    epad = jnp.full((EPAD - EE,), NP - 1, dtype=edge_index.dtype)
    rowp = jnp.concatenate([row, epad]).reshape(NW, NCH, CH)
    colp = jnp.concatenate([col, epad]).reshape(NW, NCH, CH)
    Wp = jnp.pad(W, ((0, 0), (0, F - CC)))
    bp = jnp.pad(b, (0, F - CC)).reshape(1, F)
    zeros_nf = jnp.zeros((NP, F), jnp.float32)
    ones_chf = jnp.ones((CH, F), jnp.float32)

    degp = _deg_sc(colp, ones_chf, zeros_nf)

    u0, d8 = pl.pallas_call(
        _tc_pre_body,
        out_shape=[
            jax.ShapeDtypeStruct((NP, F), jnp.float32),
            jax.ShapeDtypeStruct((NP, F), jnp.float32),
        ],
    )(x, Wp, degp)

    acc1 = _hop_sc(u0, rowp, colp, zeros_nf)

    u1 = pl.pallas_call(
        _tc_mid_body,
        out_shape=jax.ShapeDtypeStruct((NP, F), jnp.float32),
    )(acc1, u0, d8)

    acc2 = _hop_sc(u1, rowp, colp, zeros_nf)

    outp = pl.pallas_call(
        _tc_post_body,
        out_shape=jax.ShapeDtypeStruct((NP, F), jnp.float32),
    )(acc2, u1, d8, bp)

    return outp[:NN, :CC]
